# baseline (speedup 1.0000x reference)
import functools

import jax
import jax.numpy as jnp
from jax import lax
from jax.experimental import pallas as pl
from jax.experimental.pallas import tpu as pltpu
from jax.experimental.pallas import tpu_sc as plsc

_N = 10000
_E = 320000
_D = 128
_NC = 2
_NS = 16
_NW = _NC * _NS
_CHUNK = 128
_NCHUNK = 79
_EPW = _NCHUNK * _CHUNK
_EPAD = _EPW * _NW
_NPAD = 10240
_RPT = _NPAD // _NS
_NPIECE = _RPT // _CHUNK

_mesh = plsc.VectorSubcoreMesh(core_axis_name="c", subcore_axis_name="s")


def _segsum_body(feat_hbm, src_hbm, dst_hbm, rid_hbm, zrow_hbm,
                 psum_hbm,
                 sidx_v, didx_v, rows_v, acc_sp, sem):
    c = lax.axis_index("c")
    s = lax.axis_index("s")
    wid = c * _NS + s
    base = s * _RPT
    pltpu.sync_copy(zrow_hbm, rows_v)

    def zpiece(k, carry):
        off = pl.multiple_of(base + k * _CHUNK, _CHUNK)
        pltpu.sync_copy(rid_hbm.at[pl.ds(off, _CHUNK)], sidx_v)
        pltpu.sync_copy(rows_v, acc_sp.at[sidx_v])
        return carry

    lax.fori_loop(0, _NPIECE, zpiece, 0)
    plsc.subcore_barrier()

    ebase = wid * _EPW

    def chunk(j, carry):
        b = ebase + j * _CHUNK
        pltpu.sync_copy(src_hbm.at[pl.ds(b, _CHUNK)], sidx_v)
        pltpu.sync_copy(dst_hbm.at[pl.ds(b, _CHUNK)], didx_v)
        pltpu.async_copy(feat_hbm.at[sidx_v], rows_v, sem).wait()
        pltpu.sync_copy(rows_v, acc_sp.at[didx_v], add=True)
        return carry

    lax.fori_loop(0, _NCHUNK, chunk, 0)
    plsc.subcore_barrier()

    def opiece(k, carry):
        off = pl.multiple_of(base + k * _CHUNK, _CHUNK)
        pltpu.sync_copy(rid_hbm.at[pl.ds(off, _CHUNK)], sidx_v)
        pltpu.async_copy(acc_sp.at[sidx_v], rows_v, sem).wait()
        pltpu.sync_copy(rows_v, psum_hbm.at[c, pl.ds(off, _CHUNK)])
        return carry

    lax.fori_loop(0, _NPIECE, opiece, 0)


_segsum = pl.kernel(
    _segsum_body,
    out_type=jax.ShapeDtypeStruct((_NC, _NPAD, _D), jnp.float32),
    mesh=_mesh,
    scratch_types=[
        pltpu.VMEM((_CHUNK,), jnp.int32),
        pltpu.VMEM((_CHUNK,), jnp.int32),
        pltpu.VMEM((_CHUNK, _D), jnp.float32),
        pltpu.VMEM_SHARED((_NPAD, _D), jnp.float32),
        pltpu.SemaphoreType.DMA,
    ],
)


def _degcount_body(dst_hbm, rid_hbm, zrow_hbm, ones_hbm,
                   pdeg_hbm,
                   didx_v, rows_v, ones_v, acc_sp, sem):
    c = lax.axis_index("c")
    s = lax.axis_index("s")
    wid = c * _NS + s
    base = s * _RPT
    pltpu.sync_copy(zrow_hbm, rows_v)
    pltpu.sync_copy(ones_hbm, ones_v)

    def zpiece(k, carry):
        off = pl.multiple_of(base + k * _CHUNK, _CHUNK)
        pltpu.sync_copy(rid_hbm.at[pl.ds(off, _CHUNK)], didx_v)
        pltpu.sync_copy(rows_v, acc_sp.at[didx_v])
        return carry

    lax.fori_loop(0, _NPIECE, zpiece, 0)
    plsc.subcore_barrier()

    ebase = wid * _EPW

    def chunk(j, carry):
        b = ebase + j * _CHUNK
        pltpu.sync_copy(dst_hbm.at[pl.ds(b, _CHUNK)], didx_v)
        pltpu.sync_copy(ones_v, acc_sp.at[didx_v], add=True)
        return carry

    lax.fori_loop(0, _NCHUNK, chunk, 0)
    plsc.subcore_barrier()

    def opiece(k, carry):
        off = pl.multiple_of(base + k * _CHUNK, _CHUNK)
        pltpu.sync_copy(rid_hbm.at[pl.ds(off, _CHUNK)], didx_v)
        pltpu.async_copy(acc_sp.at[didx_v], rows_v, sem).wait()
        pltpu.sync_copy(rows_v, pdeg_hbm.at[c, pl.ds(off, _CHUNK)])
        return carry

    lax.fori_loop(0, _NPIECE, opiece, 0)


_degcount = pl.kernel(
    _degcount_body,
    out_type=jax.ShapeDtypeStruct((_NC, _NPAD, _D), jnp.float32),
    mesh=_mesh,
    scratch_types=[
        pltpu.VMEM((_CHUNK,), jnp.int32),
        pltpu.VMEM((_CHUNK, _D), jnp.float32),
        pltpu.VMEM((_CHUNK, _D), jnp.float32),
        pltpu.VMEM_SHARED((_NPAD, _D), jnp.float32),
        pltpu.SemaphoreType.DMA,
    ],
)


def _lane_shuffle(x, idx):
    return lax.gather(
        x, idx[:, None],
        dimension_numbers=lax.GatherDimensionNumbers(
            offset_dims=(), collapsed_slice_dims=(0,), start_index_map=(0,)),
        slice_sizes=(1,),
        mode=lax.GatherScatterMode.PROMISE_IN_BOUNDS)


def _decode_body(z_hbm, ls_hbm, ld_hbm, out_hbm,
                 aidx_v, bidx_v, arows_v, brows_v, out_v, sema, semb):
    c = lax.axis_index("c")
    s = lax.axis_index("s")
    ebase = (c * _NS + s) * _EPW

    lanes = lax.iota(jnp.int32, 16)
    xor_idx = [lanes ^ sh for sh in (1, 2, 4, 8)]

    def chunk(j, carry):
        b = ebase + j * _CHUNK
        pltpu.sync_copy(ls_hbm.at[pl.ds(b, _CHUNK)], aidx_v)
        pltpu.sync_copy(ld_hbm.at[pl.ds(b, _CHUNK)], bidx_v)
        cpa = pltpu.async_copy(z_hbm.at[aidx_v], arows_v, sema)
        cpb = pltpu.async_copy(z_hbm.at[bidx_v], brows_v, semb)
        cpa.wait()
        cpb.wait()

        for g in range(_CHUNK // 16):
            out16 = jnp.zeros((16,), jnp.float32)
            for rr in range(16):
                r = g * 16 + rr
                acc = arows_v[r, pl.ds(0, 16)] * brows_v[r, pl.ds(0, 16)]
                for dd in range(1, _D // 16):
                    acc = acc + (arows_v[r, pl.ds(dd * 16, 16)] *
                                 brows_v[r, pl.ds(dd * 16, 16)])
                for xi in xor_idx:
                    acc = acc + _lane_shuffle(acc, xi)
                out16 = jnp.where(lanes == rr, acc, out16)
            out_v[pl.ds(g * 16, 16)] = out16
        pltpu.sync_copy(out_v, out_hbm.at[pl.ds(b, _CHUNK)])
        return carry

    lax.fori_loop(0, _NCHUNK, chunk, 0)


_decode = pl.kernel(
    _decode_body,
    out_type=jax.ShapeDtypeStruct((_EPAD,), jnp.float32),
    mesh=_mesh,
    scratch_types=[
        pltpu.VMEM((_CHUNK,), jnp.int32),
        pltpu.VMEM((_CHUNK,), jnp.int32),
        pltpu.VMEM((_CHUNK, _D), jnp.float32),
        pltpu.VMEM((_CHUNK, _D), jnp.float32),
        pltpu.VMEM((_CHUNK,), jnp.float32),
        pltpu.SemaphoreType.DMA,
        pltpu.SemaphoreType.DMA,
    ],
)

_RB = 400


def _layer_block(p_ref, deg_ref, x_ref, wl_ref, wr_ref, b_ref, o_ref, *, relu):
    p = p_ref[0] + p_ref[1]
    d = jnp.maximum(deg_ref[0] + deg_ref[1], 1.0)
    agg = p / d
    acc = jnp.dot(agg, wl_ref[...], preferred_element_type=jnp.float32,
                  precision=lax.Precision.HIGHEST)
    acc = acc + jnp.dot(x_ref[...], wr_ref[...],
                        preferred_element_type=jnp.float32,
                        precision=lax.Precision.HIGHEST)
    acc = acc + b_ref[...]
    if relu:
        acc = jnp.maximum(acc, 0.0)
    o_ref[...] = acc


def _layer(psum, pdeg, x, wl, wr, b, relu):
    body = functools.partial(_layer_block, relu=relu)
    return pl.pallas_call(
        body,
        grid=(_N // _RB,),
        in_specs=[
            pl.BlockSpec((_NC, _RB, _D), lambda i: (0, i, 0)),
            pl.BlockSpec((_NC, _RB, _D), lambda i: (0, i, 0)),
            pl.BlockSpec((_RB, _D), lambda i: (i, 0)),
            pl.BlockSpec((_D, _D), lambda i: (0, 0)),
            pl.BlockSpec((_D, _D), lambda i: (0, 0)),
            pl.BlockSpec((1, _D), lambda i: (0, 0)),
        ],
        out_specs=pl.BlockSpec((_RB, _D), lambda i: (i, 0)),
        out_shape=jax.ShapeDtypeStruct((_N, _D), jnp.float32),
    )(psum, pdeg, x, wl, wr, b.reshape(1, _D))


def kernel(x, edge_index, edge_label_index, W1l, W1r, b1, W2l, W2r, b2):
    src = edge_index[0].astype(jnp.int32)
    dst = edge_index[1].astype(jnp.int32)
    lsrc = edge_label_index[0].astype(jnp.int32)
    ldst = edge_label_index[1].astype(jnp.int32)
    zrow = jnp.zeros((_CHUNK, _D), jnp.float32)
    ones = jnp.ones((_CHUNK, _D), jnp.float32)
    rid = jnp.arange(_NPAD, dtype=jnp.int32)

    pad = _EPAD - _E
    srcp = jnp.concatenate([src, jnp.zeros((pad,), jnp.int32)])
    dstp = jnp.concatenate([dst, jnp.full((pad,), _N, jnp.int32)])
    lsp = jnp.concatenate([lsrc, jnp.zeros((pad,), jnp.int32)])
    ldp = jnp.concatenate([ldst, jnp.zeros((pad,), jnp.int32)])

    degp = _degcount(dstp, rid, zrow, ones)[:, :_N]
    p1 = _segsum(x, srcp, dstp, rid, zrow)[:, :_N]
    z = _layer(p1, degp, x, W1l, W1r, b1, relu=True)
    p2 = _segsum(z, srcp, dstp, rid, zrow)[:, :_N]
    z2 = _layer(p2, degp, z, W2l, W2r, b2, relu=False)
    return _decode(z2, lsp, ldp)[:_E]


# speedup vs baseline: 1.4083x; 1.4083x over previous
import functools

import jax
import jax.numpy as jnp
from jax import lax
from jax.experimental import pallas as pl
from jax.experimental.pallas import tpu as pltpu
from jax.experimental.pallas import tpu_sc as plsc

_N = 10000
_E = 320000
_D = 128
_NC = 2
_NS = 16
_NW = _NC * _NS
_CHUNK = 80
_EPW = _E // _NW
_NCHUNK = _EPW // _CHUNK
_RPT = 640
_RPT_LAST = _N - 15 * _RPT
_DW = 16

_mesh = plsc.VectorSubcoreMesh(core_axis_name="c", subcore_axis_name="s")


_NPIECE = _RPT // _CHUNK
_NPIECE_LAST = _RPT_LAST // _CHUNK


def _segsum_body(feat_hbm, src_hbm, dst_hbm, rid_hbm, zrow_hbm,
                 psum_hbm,
                 sidx_v, didx_v, rows_v, acc_sp, sem, semi, semj):
    c = lax.axis_index("c")
    s = lax.axis_index("s")
    wid = c * _NS + s
    base = s * _RPT
    pltpu.sync_copy(zrow_hbm, rows_v)

    def zpiece(k, carry):
        @pl.when((s < 15) | (k < _NPIECE_LAST))
        def _():
            off = pl.multiple_of(base + k * _CHUNK, _CHUNK)
            pltpu.sync_copy(rid_hbm.at[pl.ds(off, _CHUNK)], sidx_v)
            pltpu.sync_copy(rows_v, acc_sp.at[sidx_v])
        return carry

    lax.fori_loop(0, _NPIECE, zpiece, 0)
    plsc.subcore_barrier()

    ebase = wid * _EPW

    def chunk(j, carry):
        b = ebase + j * _CHUNK
        ca = pltpu.async_copy(src_hbm.at[pl.ds(b, _CHUNK)], sidx_v, semi)
        cb = pltpu.async_copy(dst_hbm.at[pl.ds(b, _CHUNK)], didx_v, semj)
        ca.wait()
        cb.wait()
        pltpu.async_copy(feat_hbm.at[sidx_v], rows_v, sem).wait()
        pltpu.sync_copy(rows_v, acc_sp.at[didx_v], add=True)
        return carry

    lax.fori_loop(0, _NCHUNK, chunk, 0)
    plsc.subcore_barrier()

    def opiece(k, carry):
        @pl.when((s < 15) | (k < _NPIECE_LAST))
        def _():
            off = pl.multiple_of(base + k * _CHUNK, _CHUNK)
            pltpu.sync_copy(rid_hbm.at[pl.ds(off, _CHUNK)], sidx_v)
            pltpu.async_copy(acc_sp.at[sidx_v], rows_v, sem).wait()
            pltpu.sync_copy(rows_v, psum_hbm.at[c, pl.ds(off, _CHUNK)])
        return carry

    lax.fori_loop(0, _NPIECE, opiece, 0)


_segsum = pl.kernel(
    _segsum_body,
    out_type=jax.ShapeDtypeStruct((_NC, _N, _D), jnp.float32),
    mesh=_mesh,
    scratch_types=[
        pltpu.VMEM((_CHUNK,), jnp.int32),
        pltpu.VMEM((_CHUNK,), jnp.int32),
        pltpu.VMEM((_CHUNK, _D), jnp.float32),
        pltpu.VMEM_SHARED((_N, _D), jnp.float32),
        pltpu.SemaphoreType.DMA,
        pltpu.SemaphoreType.DMA,
        pltpu.SemaphoreType.DMA,
    ],
)


def _degcount_body(dst_hbm, rid_hbm, zrow_hbm, ones_hbm,
                   pdeg_hbm,
                   didx_v, rows_v, ones_v, acc_sp, sem):
    c = lax.axis_index("c")
    s = lax.axis_index("s")
    wid = c * _NS + s
    base = s * _RPT
    pltpu.sync_copy(zrow_hbm, rows_v)
    pltpu.sync_copy(ones_hbm, ones_v)

    def zpiece(k, carry):
        @pl.when((s < 15) | (k < _NPIECE_LAST))
        def _():
            off = pl.multiple_of(base + k * _CHUNK, _CHUNK)
            pltpu.sync_copy(rid_hbm.at[pl.ds(off, _CHUNK)], didx_v)
            pltpu.sync_copy(rows_v, acc_sp.at[didx_v])
        return carry

    lax.fori_loop(0, _NPIECE, zpiece, 0)
    plsc.subcore_barrier()

    ebase = wid * _EPW

    def chunk(j, carry):
        b = ebase + j * _CHUNK
        pltpu.sync_copy(dst_hbm.at[pl.ds(b, _CHUNK)], didx_v)
        pltpu.sync_copy(ones_v, acc_sp.at[didx_v], add=True)
        return carry

    lax.fori_loop(0, _NCHUNK, chunk, 0)
    plsc.subcore_barrier()

    def opiece(k, carry):
        @pl.when((s < 15) | (k < _NPIECE_LAST))
        def _():
            off = pl.multiple_of(base + k * _CHUNK, _CHUNK)
            pltpu.sync_copy(rid_hbm.at[pl.ds(off, _CHUNK)], didx_v)
            pltpu.async_copy(acc_sp.at[didx_v], rows_v, sem).wait()
            pltpu.sync_copy(rows_v, pdeg_hbm.at[c, pl.ds(off, _CHUNK)])
        return carry

    lax.fori_loop(0, _NPIECE, opiece, 0)


_degcount = pl.kernel(
    _degcount_body,
    out_type=jax.ShapeDtypeStruct((_NC, _N, _D), jnp.float32),
    mesh=_mesh,
    scratch_types=[
        pltpu.VMEM((_CHUNK,), jnp.int32),
        pltpu.VMEM((_CHUNK, _D), jnp.float32),
        pltpu.VMEM((_CHUNK, _D), jnp.float32),
        pltpu.VMEM_SHARED((_N, _D), jnp.float32),
        pltpu.SemaphoreType.DMA,
    ],
)


def _lane_shuffle(x, idx):
    return lax.gather(
        x, idx[:, None],
        dimension_numbers=lax.GatherDimensionNumbers(
            offset_dims=(), collapsed_slice_dims=(0,), start_index_map=(0,)),
        slice_sizes=(1,),
        mode=lax.GatherScatterMode.PROMISE_IN_BOUNDS)


def _decode_body(z_hbm, ls_hbm, ld_hbm, out_hbm,
                 aidx_v, bidx_v, arows_v, brows_v, out_v, sema, semb,
                 semi, semj):
    c = lax.axis_index("c")
    s = lax.axis_index("s")
    ebase = (c * _NS + s) * _EPW

    lanes = lax.iota(jnp.int32, 16)
    xor_idx = [lanes ^ sh for sh in (1, 2, 4, 8)]

    def chunk(j, carry):
        b = ebase + j * _CHUNK
        cia = pltpu.async_copy(ls_hbm.at[pl.ds(b, _CHUNK)], aidx_v, semi)
        cib = pltpu.async_copy(ld_hbm.at[pl.ds(b, _CHUNK)], bidx_v, semj)
        cia.wait()
        cib.wait()
        cpa = pltpu.async_copy(z_hbm.at[aidx_v], arows_v, sema)
        cpb = pltpu.async_copy(z_hbm.at[bidx_v], brows_v, semb)
        cpa.wait()
        cpb.wait()

        for g in range(_CHUNK // 16):
            out16 = jnp.zeros((16,), jnp.float32)
            for rr in range(16):
                r = g * 16 + rr
                acc = arows_v[r, pl.ds(0, 16)] * brows_v[r, pl.ds(0, 16)]
                for dd in range(1, _D // 16):
                    acc = acc + (arows_v[r, pl.ds(dd * 16, 16)] *
                                 brows_v[r, pl.ds(dd * 16, 16)])
                for xi in xor_idx:
                    acc = acc + _lane_shuffle(acc, xi)
                out16 = jnp.where(lanes == rr, acc, out16)
            out_v[pl.ds(g * 16, 16)] = out16
        pltpu.sync_copy(out_v, out_hbm.at[pl.ds(b, _CHUNK)])
        return carry

    lax.fori_loop(0, _NCHUNK, chunk, 0)


_decode = pl.kernel(
    _decode_body,
    out_type=jax.ShapeDtypeStruct((_E,), jnp.float32),
    mesh=_mesh,
    scratch_types=[
        pltpu.VMEM((_CHUNK,), jnp.int32),
        pltpu.VMEM((_CHUNK,), jnp.int32),
        pltpu.VMEM((_CHUNK, _D), jnp.float32),
        pltpu.VMEM((_CHUNK, _D), jnp.float32),
        pltpu.VMEM((_CHUNK,), jnp.float32),
        pltpu.SemaphoreType.DMA,
        pltpu.SemaphoreType.DMA,
        pltpu.SemaphoreType.DMA,
        pltpu.SemaphoreType.DMA,
    ],
)

_RB = 400


def _layer_block(p_ref, deg_ref, x_ref, wl_ref, wr_ref, b_ref, o_ref, *, relu):
    p = p_ref[0] + p_ref[1]
    d = jnp.maximum(deg_ref[0] + deg_ref[1], 1.0)
    agg = p / d
    acc = jnp.dot(agg, wl_ref[...], preferred_element_type=jnp.float32,
                  precision=lax.Precision.HIGHEST)
    acc = acc + jnp.dot(x_ref[...], wr_ref[...],
                        preferred_element_type=jnp.float32,
                        precision=lax.Precision.HIGHEST)
    acc = acc + b_ref[...]
    if relu:
        acc = jnp.maximum(acc, 0.0)
    o_ref[...] = acc


def _layer(psum, pdeg, x, wl, wr, b, relu):
    body = functools.partial(_layer_block, relu=relu)
    return pl.pallas_call(
        body,
        grid=(_N // _RB,),
        in_specs=[
            pl.BlockSpec((_NC, _RB, _D), lambda i: (0, i, 0)),
            pl.BlockSpec((_NC, _RB, _D), lambda i: (0, i, 0)),
            pl.BlockSpec((_RB, _D), lambda i: (i, 0)),
            pl.BlockSpec((_D, _D), lambda i: (0, 0)),
            pl.BlockSpec((_D, _D), lambda i: (0, 0)),
            pl.BlockSpec((1, _D), lambda i: (0, 0)),
        ],
        out_specs=pl.BlockSpec((_RB, _D), lambda i: (i, 0)),
        out_shape=jax.ShapeDtypeStruct((_N, _D), jnp.float32),
    )(psum, pdeg, x, wl, wr, b.reshape(1, _D))


def kernel(x, edge_index, edge_label_index, W1l, W1r, b1, W2l, W2r, b2):
    src = edge_index[0].astype(jnp.int32)
    dst = edge_index[1].astype(jnp.int32)
    lsrc = edge_label_index[0].astype(jnp.int32)
    ldst = edge_label_index[1].astype(jnp.int32)
    zrow = jnp.zeros((_CHUNK, _D), jnp.float32)
    ones = jnp.ones((_CHUNK, _D), jnp.float32)
    rid = jnp.arange(_N, dtype=jnp.int32)

    degp = _degcount(dst, rid, zrow, ones)
    p1 = _segsum(x, src, dst, rid, zrow)
    z = _layer(p1, degp, x, W1l, W1r, b1, relu=True)
    p2 = _segsum(z, src, dst, rid, zrow)
    z2 = _layer(p2, degp, z, W2l, W2r, b2, relu=False)
    return _decode(z2, lsrc, ldst)


# speedup vs baseline: 1.6352x; 1.1612x over previous
import functools

import jax
import jax.numpy as jnp
from jax import lax
from jax.experimental import pallas as pl
from jax.experimental.pallas import tpu as pltpu
from jax.experimental.pallas import tpu_sc as plsc

_N = 10000
_E = 320000
_D = 128
_NC = 2
_NS = 16
_NW = _NC * _NS
_CHUNK = 80
_EPW = _E // _NW
_NCHUNK = _EPW // _CHUNK
_RPT = 640
_RPT_LAST = _N - 15 * _RPT
_DW = 16

_mesh = plsc.VectorSubcoreMesh(core_axis_name="c", subcore_axis_name="s")


_NPIECE = _RPT // _CHUNK
_NPIECE_LAST = _RPT_LAST // _CHUNK


def _segsum_body(feat_hbm, src_hbm, dst_hbm, rid_hbm, zrow_hbm,
                 psum_hbm,
                 sidx_v, didx_v, rows_v, acc_sp, sem, semi, semj):
    c = lax.axis_index("c")
    s = lax.axis_index("s")
    wid = c * _NS + s
    base = s * _RPT
    pltpu.sync_copy(zrow_hbm, rows_v)

    def zpiece(k, carry):
        @pl.when((s < 15) | (k < _NPIECE_LAST))
        def _():
            off = pl.multiple_of(base + k * _CHUNK, _CHUNK)
            pltpu.sync_copy(rid_hbm.at[pl.ds(off, _CHUNK)], sidx_v)
            pltpu.sync_copy(rows_v, acc_sp.at[sidx_v])
        return carry

    lax.fori_loop(0, _NPIECE, zpiece, 0)
    plsc.subcore_barrier()

    ebase = wid * _EPW
    pltpu.sync_copy(src_hbm.at[pl.ds(ebase, _CHUNK)], sidx_v)

    def chunk(j, carry):
        b = ebase + j * _CHUNK
        cb = pltpu.async_copy(dst_hbm.at[pl.ds(b, _CHUNK)], didx_v, semj)
        cg = pltpu.async_copy(feat_hbm.at[sidx_v], rows_v, sem)
        cg.wait()
        nb = pl.multiple_of(
            jnp.where(j + 1 < _NCHUNK, b + _CHUNK, ebase), _CHUNK)
        ci = pltpu.async_copy(src_hbm.at[pl.ds(nb, _CHUNK)], sidx_v, semi)
        cb.wait()
        pltpu.sync_copy(rows_v, acc_sp.at[didx_v], add=True)
        ci.wait()
        return carry

    lax.fori_loop(0, _NCHUNK, chunk, 0)
    plsc.subcore_barrier()

    def opiece(k, carry):
        @pl.when((s < 15) | (k < _NPIECE_LAST))
        def _():
            off = pl.multiple_of(base + k * _CHUNK, _CHUNK)
            pltpu.sync_copy(rid_hbm.at[pl.ds(off, _CHUNK)], sidx_v)
            pltpu.async_copy(acc_sp.at[sidx_v], rows_v, sem).wait()
            pltpu.sync_copy(rows_v, psum_hbm.at[c, pl.ds(off, _CHUNK)])
        return carry

    lax.fori_loop(0, _NPIECE, opiece, 0)


_segsum = pl.kernel(
    _segsum_body,
    out_type=jax.ShapeDtypeStruct((_NC, _N, _D), jnp.float32),
    mesh=_mesh,
    scratch_types=[
        pltpu.VMEM((_CHUNK,), jnp.int32),
        pltpu.VMEM((_CHUNK,), jnp.int32),
        pltpu.VMEM((_CHUNK, _D), jnp.float32),
        pltpu.VMEM_SHARED((_N, _D), jnp.float32),
        pltpu.SemaphoreType.DMA,
        pltpu.SemaphoreType.DMA,
        pltpu.SemaphoreType.DMA,
    ],
)


def _degcount_body(dst_hbm, rid_hbm, zrow_hbm, ones_hbm,
                   pdeg_hbm,
                   didx_v, didx2_v, rows_v, ones_v, acc_sp, sem, sema, semb):
    c = lax.axis_index("c")
    s = lax.axis_index("s")
    wid = c * _NS + s
    base = s * _RPT
    pltpu.sync_copy(zrow_hbm, rows_v)
    pltpu.sync_copy(ones_hbm, ones_v)

    def zpiece(k, carry):
        @pl.when((s < 15) | (k < _NPIECE_LAST))
        def _():
            off = pl.multiple_of(base + k * _CHUNK, _CHUNK)
            pltpu.sync_copy(rid_hbm.at[pl.ds(off, _CHUNK)], didx_v)
            pltpu.sync_copy(rows_v, acc_sp.at[didx_v])
        return carry

    lax.fori_loop(0, _NPIECE, zpiece, 0)
    plsc.subcore_barrier()

    ebase = wid * _EPW
    pltpu.sync_copy(dst_hbm.at[pl.ds(ebase, _CHUNK)], didx_v)

    def chunkpair(jj, carry):
        b0 = ebase + jj * (2 * _CHUNK)
        cb1 = pltpu.async_copy(
            dst_hbm.at[pl.ds(b0 + _CHUNK, _CHUNK)], didx2_v, semb)
        pltpu.sync_copy(ones_v, acc_sp.at[didx_v], add=True)
        nb = pl.multiple_of(
            jnp.where(b0 + 2 * _CHUNK < ebase + _EPW, b0 + 2 * _CHUNK,
                      ebase), _CHUNK)
        cb2 = pltpu.async_copy(dst_hbm.at[pl.ds(nb, _CHUNK)], didx_v, sema)
        cb1.wait()
        pltpu.sync_copy(ones_v, acc_sp.at[didx2_v], add=True)
        cb2.wait()
        return carry

    lax.fori_loop(0, _NCHUNK // 2, chunkpair, 0)
    pltpu.sync_copy(ones_v, acc_sp.at[didx_v], add=True)
    plsc.subcore_barrier()

    def opiece(k, carry):
        @pl.when((s < 15) | (k < _NPIECE_LAST))
        def _():
            off = pl.multiple_of(base + k * _CHUNK, _CHUNK)
            pltpu.sync_copy(rid_hbm.at[pl.ds(off, _CHUNK)], didx_v)
            pltpu.async_copy(acc_sp.at[didx_v], rows_v, sem).wait()
            pltpu.sync_copy(rows_v, pdeg_hbm.at[c, pl.ds(off, _CHUNK)])
        return carry

    lax.fori_loop(0, _NPIECE, opiece, 0)


_degcount = pl.kernel(
    _degcount_body,
    out_type=jax.ShapeDtypeStruct((_NC, _N, _D), jnp.float32),
    mesh=_mesh,
    scratch_types=[
        pltpu.VMEM((_CHUNK,), jnp.int32),
        pltpu.VMEM((_CHUNK,), jnp.int32),
        pltpu.VMEM((_CHUNK, _D), jnp.float32),
        pltpu.VMEM((_CHUNK, _D), jnp.float32),
        pltpu.VMEM_SHARED((_N, _D), jnp.float32),
        pltpu.SemaphoreType.DMA,
        pltpu.SemaphoreType.DMA,
        pltpu.SemaphoreType.DMA,
    ],
)


def _lane_shuffle(x, idx):
    return lax.gather(
        x, idx[:, None],
        dimension_numbers=lax.GatherDimensionNumbers(
            offset_dims=(), collapsed_slice_dims=(0,), start_index_map=(0,)),
        slice_sizes=(1,),
        mode=lax.GatherScatterMode.PROMISE_IN_BOUNDS)


def _decode_body(z_hbm, ls_hbm, ld_hbm, out_hbm,
                 aidx_v, bidx_v, arows_v, brows_v, out_v, sema, semb,
                 semi, semj):
    c = lax.axis_index("c")
    s = lax.axis_index("s")
    ebase = (c * _NS + s) * _EPW

    lanes = lax.iota(jnp.int32, 16)
    xor_idx = [lanes ^ sh for sh in (1, 2, 4, 8)]

    pltpu.sync_copy(ls_hbm.at[pl.ds(ebase, _CHUNK)], aidx_v)
    pltpu.sync_copy(ld_hbm.at[pl.ds(ebase, _CHUNK)], bidx_v)

    def chunk(j, carry):
        b = ebase + j * _CHUNK
        cpa = pltpu.async_copy(z_hbm.at[aidx_v], arows_v, sema)
        cpb = pltpu.async_copy(z_hbm.at[bidx_v], brows_v, semb)
        cpa.wait()
        cpb.wait()
        nb = pl.multiple_of(
            jnp.where(j + 1 < _NCHUNK, b + _CHUNK, ebase), _CHUNK)
        cia = pltpu.async_copy(ls_hbm.at[pl.ds(nb, _CHUNK)], aidx_v, semi)
        cib = pltpu.async_copy(ld_hbm.at[pl.ds(nb, _CHUNK)], bidx_v, semj)

        for g in range(_CHUNK // 16):
            out16 = jnp.zeros((16,), jnp.float32)
            for rr in range(16):
                r = g * 16 + rr
                acc = arows_v[r, pl.ds(0, 16)] * brows_v[r, pl.ds(0, 16)]
                for dd in range(1, _D // 16):
                    acc = acc + (arows_v[r, pl.ds(dd * 16, 16)] *
                                 brows_v[r, pl.ds(dd * 16, 16)])
                for xi in xor_idx:
                    acc = acc + _lane_shuffle(acc, xi)
                out16 = jnp.where(lanes == rr, acc, out16)
            out_v[pl.ds(g * 16, 16)] = out16
        pltpu.sync_copy(out_v, out_hbm.at[pl.ds(b, _CHUNK)])
        cia.wait()
        cib.wait()
        return carry

    lax.fori_loop(0, _NCHUNK, chunk, 0)


_decode = pl.kernel(
    _decode_body,
    out_type=jax.ShapeDtypeStruct((_E,), jnp.float32),
    mesh=_mesh,
    scratch_types=[
        pltpu.VMEM((_CHUNK,), jnp.int32),
        pltpu.VMEM((_CHUNK,), jnp.int32),
        pltpu.VMEM((_CHUNK, _D), jnp.float32),
        pltpu.VMEM((_CHUNK, _D), jnp.float32),
        pltpu.VMEM((_CHUNK,), jnp.float32),
        pltpu.SemaphoreType.DMA,
        pltpu.SemaphoreType.DMA,
        pltpu.SemaphoreType.DMA,
        pltpu.SemaphoreType.DMA,
    ],
)

_RB = 400


def _layer_block(p_ref, deg_ref, x_ref, wl_ref, wr_ref, b_ref, o_ref, *, relu):
    p = p_ref[0] + p_ref[1]
    d = jnp.maximum(deg_ref[0] + deg_ref[1], 1.0)
    agg = p / d
    acc = jnp.dot(agg, wl_ref[...], preferred_element_type=jnp.float32,
                  precision=lax.Precision.HIGHEST)
    acc = acc + jnp.dot(x_ref[...], wr_ref[...],
                        preferred_element_type=jnp.float32,
                        precision=lax.Precision.HIGHEST)
    acc = acc + b_ref[...]
    if relu:
        acc = jnp.maximum(acc, 0.0)
    o_ref[...] = acc


def _layer(psum, pdeg, x, wl, wr, b, relu):
    body = functools.partial(_layer_block, relu=relu)
    return pl.pallas_call(
        body,
        grid=(_N // _RB,),
        in_specs=[
            pl.BlockSpec((_NC, _RB, _D), lambda i: (0, i, 0)),
            pl.BlockSpec((_NC, _RB, _D), lambda i: (0, i, 0)),
            pl.BlockSpec((_RB, _D), lambda i: (i, 0)),
            pl.BlockSpec((_D, _D), lambda i: (0, 0)),
            pl.BlockSpec((_D, _D), lambda i: (0, 0)),
            pl.BlockSpec((1, _D), lambda i: (0, 0)),
        ],
        out_specs=pl.BlockSpec((_RB, _D), lambda i: (i, 0)),
        out_shape=jax.ShapeDtypeStruct((_N, _D), jnp.float32),
    )(psum, pdeg, x, wl, wr, b.reshape(1, _D))


def kernel(x, edge_index, edge_label_index, W1l, W1r, b1, W2l, W2r, b2):
    src = edge_index[0].astype(jnp.int32)
    dst = edge_index[1].astype(jnp.int32)
    lsrc = edge_label_index[0].astype(jnp.int32)
    ldst = edge_label_index[1].astype(jnp.int32)
    zrow = jnp.zeros((_CHUNK, _D), jnp.float32)
    ones = jnp.ones((_CHUNK, _D), jnp.float32)
    rid = jnp.arange(_N, dtype=jnp.int32)

    degp = _degcount(dst, rid, zrow, ones)
    p1 = _segsum(x, src, dst, rid, zrow)
    z = _layer(p1, degp, x, W1l, W1r, b1, relu=True)
    p2 = _segsum(z, src, dst, rid, zrow)
    z2 = _layer(p2, degp, z, W2l, W2r, b2, relu=False)
    return _decode(z2, lsrc, ldst)
